# Initial kernel scaffold; baseline (speedup 1.0000x reference)
import jax
import jax.numpy as jnp
from jax.experimental import pallas as pl


def kernel(x, TokenEmbeddings):
    raise NotImplementedError("write your pallas kernel here")


# speedup vs baseline: 1.8699x; 1.8699x over previous
import functools

import jax
import jax.numpy as jnp
from jax import lax
from jax.experimental import pallas as pl
from jax.experimental.pallas import tpu as pltpu
from jax.experimental.pallas import tpu_sc as plsc

NC = 2
NS = 16
NW = NC * NS

IDXROW = 128
ROWS_PER_CHUNK = 256
NBUF = 4


@functools.lru_cache(maxsize=None)
def _build(vocab, emb, total):
    per_w = total // NW
    nrows = per_w // IDXROW
    chunks = per_w // ROWS_PER_CHUNK
    kpc = ROWS_PER_CHUNK // IDXROW
    n_outer = chunks // NBUF

    mesh = plsc.VectorSubcoreMesh(core_axis_name="c", subcore_axis_name="s")

    @functools.partial(
        pl.kernel,
        out_type=jax.ShapeDtypeStruct((total, emb), jnp.float32),
        mesh=mesh,
        scratch_types=[
            pltpu.VMEM((nrows, IDXROW), jnp.int32),
            pltpu.VMEM((NBUF, ROWS_PER_CHUNK, emb), jnp.float32),
            [pltpu.SemaphoreType.DMA] * NBUF,
            [pltpu.SemaphoreType.DMA] * NBUF,
        ],
        compiler_params=pltpu.CompilerParams(use_tc_tiling_on_sc=False),
    )
    def emb_kernel(table_hbm, idx_hbm, out_hbm, idx_v, rows_v, gsems, wsems):
        wid = lax.axis_index("s") * NC + lax.axis_index("c")
        base = wid * per_w

        pltpu.sync_copy(idx_hbm.at[wid], idx_v)

        def write_wait(b):
            pltpu.make_async_copy(
                rows_v.at[b],
                out_hbm.at[pl.ds(0, ROWS_PER_CHUNK)],
                wsems[b],
            ).wait()

        def outer(c0, carry):
            handles = []
            for b in range(NBUF):
                @pl.when(c0 > 0)
                def _(b=b):
                    write_wait(b)

                ch = c0 * NBUF + b
                hs = []
                for j in range(kpc):
                    hs.append(pltpu.async_copy(
                        table_hbm.at[idx_v.at[ch * kpc + j]],
                        rows_v.at[b, pl.ds(j * IDXROW, IDXROW)],
                        gsems[b],
                    ))
                handles.append(hs)
            for b in range(NBUF):
                for h in handles[b]:
                    h.wait()
                ch = c0 * NBUF + b
                pltpu.async_copy(
                    rows_v.at[b],
                    out_hbm.at[pl.ds(base + ch * ROWS_PER_CHUNK,
                                     ROWS_PER_CHUNK)],
                    wsems[b],
                )
            return carry

        lax.fori_loop(0, n_outer, outer, 0, unroll=False)
        for b in range(NBUF):
            write_wait(b)

    return emb_kernel


def kernel(x, TokenEmbeddings):
    b, l = x.shape
    vocab, emb = TokenEmbeddings.shape
    total = b * l
    idx = x.reshape(total).astype(jnp.int32)
    idx3 = idx.reshape(NW, total // NW // IDXROW, IDXROW)
    out = _build(vocab, emb, total)(TokenEmbeddings, idx3)
    return out.reshape(b, l, emb)
